# baseline (speedup 1.0000x reference)
import dataclasses
import functools

import jax
import jax.numpy as jnp
from jax import lax
from jax.experimental import pallas as pl
from jax.experimental.pallas import tpu as pltpu
from jax.experimental.pallas import tpu_sc as plsc

NUM_CORES = 2
NUM_SUBCORES = 16
NW = NUM_CORES * NUM_SUBCORES
LANES = 16

BATCH = 16384
NFEAT = 100000
DIM = 64
B_PER_W = BATCH // NW

TCH = 256
NFULL = NFEAT // TCH
TAIL = NFEAT - NFULL * TCH
SLOTS = -(-NFULL // NW)


def _make_params(tc_tiling):
    cp = pltpu.CompilerParams()
    for fld, val in (("needs_layout_passes", False),
                     ("use_tc_tiling_on_sc", tc_tiling)):
        if fld in pltpu.CompilerParams.__dataclass_fields__:
            cp = dataclasses.replace(cp, **{fld: val})
    return cp


def _tr_kernel(tt_hbm, tail_hbm, flat_hbm, inA, inB, flatA, flatB,
               siA, siB, soA, soB):
    wid = lax.axis_index("s") * NUM_CORES + lax.axis_index("c")
    ins = (inA, inB)
    flats = (flatA, flatB)
    isems = (siA, siB)
    osems = (soA, soB)
    lane_iota = lax.iota(jnp.int32, LANES)

    def chunk_col(k):
        raw = wid + k * NW
        c = jnp.where(raw < NFULL, raw, wid)
        return c * TCH

    def issue_in(k):
        buf = k % 2
        col0 = chunk_col(k)
        return pltpu.async_copy(tt_hbm.at[:, pl.ds(col0, TCH)],
                                ins[buf], isems[buf])

    def transpose_block(in_v, flat_v, ncols):
        @pl.loop(0, ncols, step=LANES)
        def _(i0):
            for i in range(LANES):
                col = i0 + i
                col_splat = jnp.full((LANES,), 0, jnp.int32) + col
                for d0 in range(0, DIM, LANES):
                    vec = plsc.load_gather(in_v, [d0 + lane_iota, col_splat])
                    flat_v[pl.ds(col * DIM + d0, LANES)] = vec

    out_cp = [None, None]
    in_cp = [None, None]
    in_cp[0] = issue_in(0)
    for k in range(SLOTS):
        buf = k % 2
        if k + 1 < SLOTS:
            in_cp[1 - buf] = issue_in(k + 1)
        in_cp[buf].wait()
        if out_cp[buf] is not None:
            out_cp[buf].wait()
        transpose_block(ins[buf], flats[buf], TCH)
        out_cp[buf] = pltpu.async_copy(
            flats[buf], flat_hbm.at[pl.ds(chunk_col(k) * DIM, TCH * DIM)],
            osems[buf])
    out_cp[0].wait()
    out_cp[1].wait()

    @pl.when(wid == NW - 1)
    def _():
        pltpu.sync_copy(tail_hbm, flatA.at[pl.ds(0, TAIL * DIM)])
        pltpu.sync_copy(flatA.at[pl.ds(0, TAIL * DIM)],
                        flat_hbm.at[pl.ds(NFULL * TCH * DIM, TAIL * DIM)])


def _mf_kernel(ids_hbm, vals_hbm, table_hbm, out_hbm,
               idx0_v, idx1_v, rows0_v, rows1_v, v0_v, v1_v, out_v, part_v,
               sem0, sem1):
    wid = lax.axis_index("s") * NUM_CORES + lax.axis_index("c")
    base = wid * B_PER_W

    pltpu.sync_copy(ids_hbm.at[0, pl.ds(base, B_PER_W)], idx0_v)
    pltpu.sync_copy(ids_hbm.at[1, pl.ds(base, B_PER_W)], idx1_v)
    pltpu.sync_copy(vals_hbm.at[0, pl.ds(base, B_PER_W)], v0_v)
    pltpu.sync_copy(vals_hbm.at[1, pl.ds(base, B_PER_W)], v1_v)

    cp0 = pltpu.async_copy(table_hbm.at[idx0_v], rows0_v, sem0)
    cp1 = pltpu.async_copy(table_hbm.at[idx1_v], rows1_v, sem1)
    cp0.wait()
    cp1.wait()

    lane_iota = lax.iota(jnp.int32, LANES)
    zeros16 = jnp.full((LANES,), 0, jnp.int32)

    @pl.loop(0, B_PER_W, step=LANES)
    def _(g):
        for r in range(LANES):
            b = g + r
            part = rows0_v[b, pl.ds(0, LANES)] * rows1_v[b, pl.ds(0, LANES)]
            for d in range(LANES, DIM, LANES):
                part += rows0_v[b, pl.ds(d, LANES)] * rows1_v[b, pl.ds(d, LANES)]
            part_v[r, pl.ds(0, LANES)] = part
        acc = plsc.load_gather(part_v, [lane_iota, zeros16])
        for c in range(1, LANES):
            acc += plsc.load_gather(part_v, [lane_iota,
                                             jnp.full((LANES,), c, jnp.int32)])
        sl = pl.ds(g, LANES)
        out_v[sl] = acc * v0_v[sl] * v1_v[sl]

    pltpu.sync_copy(out_v, out_hbm.at[pl.ds(base, B_PER_W)])


@jax.jit
def kernel(feature_ids, feature_vals, table):
    ids_t = feature_ids.T
    vals_t = feature_vals.T
    table_t = table.T
    tail_flat = table[NFULL * TCH:, :].reshape(-1)

    mesh = plsc.VectorSubcoreMesh(core_axis_name="c", subcore_axis_name="s")

    tr = functools.partial(
        pl.kernel,
        mesh=mesh,
        compiler_params=_make_params(True),
        out_type=jax.ShapeDtypeStruct((NFEAT * DIM,), jnp.float32),
        scratch_types=[
            pltpu.VMEM((DIM, TCH), jnp.float32),
            pltpu.VMEM((DIM, TCH), jnp.float32),
            pltpu.VMEM((TCH * DIM,), jnp.float32),
            pltpu.VMEM((TCH * DIM,), jnp.float32),
            pltpu.SemaphoreType.DMA,
            pltpu.SemaphoreType.DMA,
            pltpu.SemaphoreType.DMA,
            pltpu.SemaphoreType.DMA,
        ],
    )(_tr_kernel)
    table_rm = tr(table_t, tail_flat).reshape(NFEAT, DIM)

    run = functools.partial(
        pl.kernel,
        mesh=mesh,
        compiler_params=_make_params(False),
        out_type=jax.ShapeDtypeStruct((BATCH,), jnp.float32),
        scratch_types=[
            pltpu.VMEM((B_PER_W,), jnp.int32),
            pltpu.VMEM((B_PER_W,), jnp.int32),
            pltpu.VMEM((B_PER_W, DIM), jnp.float32),
            pltpu.VMEM((B_PER_W, DIM), jnp.float32),
            pltpu.VMEM((B_PER_W,), jnp.float32),
            pltpu.VMEM((B_PER_W,), jnp.float32),
            pltpu.VMEM((B_PER_W,), jnp.float32),
            pltpu.VMEM((LANES, LANES), jnp.float32),
            pltpu.SemaphoreType.DMA,
            pltpu.SemaphoreType.DMA,
        ],
    )(_mf_kernel)
    return run(ids_t, vals_t, table_rm)


# speedup vs baseline: 2.3523x; 2.3523x over previous
import dataclasses
import functools

import jax
import jax.numpy as jnp
from jax import lax
from jax.experimental import pallas as pl
from jax.experimental.pallas import tpu as pltpu
from jax.experimental.pallas import tpu_sc as plsc

NUM_CORES = 2
NUM_SUBCORES = 16
NW = NUM_CORES * NUM_SUBCORES
LANES = 16

BATCH = 16384
DIM = 64
WIDE = 2 * DIM
B_PER_W = BATCH // NW
CHUNK = 128
NCHUNK = B_PER_W // CHUNK


def _mf_kernel(ids_hbm, vals_hbm, table_hbm, out_hbm,
               idx0_v, idx1_v, v0_v, v1_v, out_v, part_v,
               rows0a_v, rows0b_v, rows1a_v, rows1b_v,
               sem0a, sem0b, sem1a, sem1b):
    wid = lax.axis_index("s") * NUM_CORES + lax.axis_index("c")
    base = wid * B_PER_W

    pltpu.sync_copy(ids_hbm.at[0, pl.ds(base, B_PER_W)], idx0_v)
    pltpu.sync_copy(ids_hbm.at[1, pl.ds(base, B_PER_W)], idx1_v)
    pltpu.sync_copy(vals_hbm.at[0, pl.ds(base, B_PER_W)], v0_v)
    pltpu.sync_copy(vals_hbm.at[1, pl.ds(base, B_PER_W)], v1_v)

    rows0 = (rows0a_v, rows0b_v)
    rows1 = (rows1a_v, rows1b_v)
    sems0 = (sem0a, sem0b)
    sems1 = (sem1a, sem1b)

    def issue(c):
        lo = c * CHUNK
        buf = c % 2
        return (
            pltpu.async_copy(table_hbm.at[idx0_v.at[pl.ds(lo, CHUNK)]],
                             rows0[buf], sems0[buf]),
            pltpu.async_copy(table_hbm.at[idx1_v.at[pl.ds(lo, CHUNK)]],
                             rows1[buf], sems1[buf]),
        )

    lane_iota = lax.iota(jnp.int32, LANES)
    zeros16 = jnp.full((LANES,), 0, jnp.int32)

    inflight = {0: issue(0)}
    for c in range(NCHUNK):
        if c + 1 < NCHUNK:
            inflight[c + 1] = issue(c + 1)
        cp0, cp1 = inflight.pop(c)
        cp0.wait()
        cp1.wait()
        buf = c % 2
        r0_v = rows0[buf]
        r1_v = rows1[buf]
        clo = c * CHUNK

        @pl.loop(0, CHUNK, step=LANES)
        def _(g):
            for r in range(LANES):
                b = g + r
                part = (r0_v[b, pl.ds(0, LANES)]
                        * r1_v[b, pl.ds(0, LANES)])
                for d in range(LANES, DIM, LANES):
                    part += (r0_v[b, pl.ds(d, LANES)]
                             * r1_v[b, pl.ds(d, LANES)])
                part_v[r, pl.ds(0, LANES)] = part
            acc = plsc.load_gather(part_v, [lane_iota, zeros16])
            for cc in range(1, LANES):
                acc += plsc.load_gather(
                    part_v, [lane_iota, jnp.full((LANES,), cc, jnp.int32)])
            sl = pl.ds(clo + g, LANES)
            out_v[sl] = acc * v0_v[sl] * v1_v[sl]

    pltpu.sync_copy(out_v, out_hbm.at[pl.ds(base, B_PER_W)])


@jax.jit
def kernel(feature_ids, feature_vals, table):
    ids_t = feature_ids.T
    vals_t = feature_vals.T
    table_w = jnp.pad(table, ((0, 0), (0, DIM)))

    mesh = plsc.VectorSubcoreMesh(core_axis_name="c", subcore_axis_name="s")
    cp = pltpu.CompilerParams()
    for fld, val in (("needs_layout_passes", False),
                     ("use_tc_tiling_on_sc", False)):
        if fld in pltpu.CompilerParams.__dataclass_fields__:
            cp = dataclasses.replace(cp, **{fld: val})
    run = functools.partial(
        pl.kernel,
        mesh=mesh,
        compiler_params=cp,
        out_type=jax.ShapeDtypeStruct((BATCH,), jnp.float32),
        scratch_types=[
            pltpu.VMEM((B_PER_W,), jnp.int32),
            pltpu.VMEM((B_PER_W,), jnp.int32),
            pltpu.VMEM((B_PER_W,), jnp.float32),
            pltpu.VMEM((B_PER_W,), jnp.float32),
            pltpu.VMEM((B_PER_W,), jnp.float32),
            pltpu.VMEM((LANES, LANES), jnp.float32),
            pltpu.VMEM((CHUNK, WIDE), jnp.float32),
            pltpu.VMEM((CHUNK, WIDE), jnp.float32),
            pltpu.VMEM((CHUNK, WIDE), jnp.float32),
            pltpu.VMEM((CHUNK, WIDE), jnp.float32),
            pltpu.SemaphoreType.DMA,
            pltpu.SemaphoreType.DMA,
            pltpu.SemaphoreType.DMA,
            pltpu.SemaphoreType.DMA,
        ],
    )(_mf_kernel)
    return run(ids_t, vals_t, table_w)
